# baseline (speedup 1.0000x reference)
import functools

import jax
import jax.numpy as jnp
from jax import lax
from jax.experimental import pallas as pl
from jax.experimental.pallas import tpu as pltpu
from jax.experimental.pallas import tpu_sc as plsc

_ROWS = 8192
_FEAT = 256
_NC = 2
_NS = 16
_NW = _NC * _NS
_ROWS_PER_W = _ROWS // _NW


def _sc_copy(src_hbm, out_hbm, buf):
    wid = lax.axis_index("s") * _NC + lax.axis_index("c")
    base = wid * _ROWS_PER_W
    pltpu.sync_copy(src_hbm.at[pl.ds(base, _ROWS_PER_W)], buf)
    pltpu.sync_copy(buf, out_hbm.at[pl.ds(base, _ROWS_PER_W)])


def kernel(prototypes):
    mesh = plsc.VectorSubcoreMesh(core_axis_name="c", subcore_axis_name="s")
    k = functools.partial(
        pl.kernel,
        mesh=mesh,
        out_type=jax.ShapeDtypeStruct((_ROWS, _FEAT), jnp.float32),
        scratch_types=[pltpu.VMEM((_ROWS_PER_W, _FEAT), jnp.float32)],
    )(_sc_copy)
    return k(prototypes)
